# Initial kernel scaffold; baseline (speedup 1.0000x reference)
import jax
import jax.numpy as jnp
from jax.experimental import pallas as pl
from jax.experimental.pallas import tpu as pltpu
from jax.experimental.pallas import tpu_sc as plsc

_WINDOW = 128


def kernel(token_ids, weights):
    batch, seq = token_ids.shape
    num_idx = batch * seq
    dim = weights.shape[1]
    indices = token_ids.reshape(1, num_idx)

    mesh = plsc.VectorSubcoreMesh(
        core_axis_name="core", subcore_axis_name="subcore"
    )

    @pl.kernel(
        out_type=jax.ShapeDtypeStruct((num_idx, dim), weights.dtype),
        mesh=mesh,
    )
    def gather_kernel(x_hbm, i_hbm, o_hbm):
        def body(i_vmem, o_vmem):
            pltpu.sync_copy(x_hbm.at[i_vmem.at[0]], o_vmem)

        pltpu.emit_pipeline(
            body,
            grid=(num_idx // _WINDOW,),
            in_specs=[
                pl.BlockSpec((1, _WINDOW), index_map=lambda i: (0, i))
            ],
            out_specs=[
                pl.BlockSpec((_WINDOW, dim), index_map=lambda i: (i, 0))
            ],
            core_axis_name=("core", "subcore"),
            dimension_semantics=(pltpu.PARALLEL,),
        )(i_hbm, o_hbm)

    out = gather_kernel(weights, indices)
    return out.reshape(batch, seq, dim)


# speedup vs baseline: 1.0388x; 1.0388x over previous
import functools

import jax
import jax.numpy as jnp
from jax import lax
from jax.experimental import pallas as pl
from jax.experimental.pallas import tpu as pltpu
from jax.experimental.pallas import tpu_sc as plsc

_NUM_CORES = 2
_NUM_SUBCORES = 16
_NUM_WORKERS = _NUM_CORES * _NUM_SUBCORES
_CHUNK = 128


def kernel(token_ids, weights):
    batch, seq = token_ids.shape
    num_idx = batch * seq
    num_rows, dim = weights.shape

    idx = token_ids.reshape(num_idx)
    half = lax.shift_right_logical(idx, 1)
    wpair = weights.reshape(num_rows // 2, 2 * dim)

    b_per_w = num_idx // _NUM_WORKERS
    n_chunks = b_per_w // _CHUNK

    mesh = plsc.VectorSubcoreMesh(core_axis_name="c", subcore_axis_name="s")

    @functools.partial(
        pl.kernel,
        mesh=mesh,
        out_type=jax.ShapeDtypeStruct((num_idx, 2 * dim), weights.dtype),
        scratch_types=[
            pltpu.VMEM((_CHUNK,), jnp.int32),
            pltpu.VMEM((_CHUNK, 2 * dim), jnp.float32),
            pltpu.SemaphoreType.DMA,
        ],
    )
    def gather_kernel(table_hbm, ih_hbm, out_hbm, ih_v, rows_v, sem):
        wid = lax.axis_index("s") * _NUM_CORES + lax.axis_index("c")
        base = wid * b_per_w

        @pl.loop(0, n_chunks)
        def _(c):
            off = base + c * _CHUNK
            pltpu.sync_copy(ih_hbm.at[pl.ds(off, _CHUNK)], ih_v)
            pltpu.async_copy(table_hbm.at[ih_v], rows_v, sem).wait()
            pltpu.sync_copy(rows_v, out_hbm.at[pl.ds(off, _CHUNK)])

    pairs = gather_kernel(wpair, half)
    odd = lax.bitwise_and(idx, 1)[:, None] == 1
    out = jnp.where(odd, pairs[:, dim:], pairs[:, :dim])
    return out.reshape(batch, seq, dim)


# speedup vs baseline: 1.3345x; 1.2847x over previous
import functools

import jax
import jax.numpy as jnp
from jax import lax
from jax.experimental import pallas as pl
from jax.experimental.pallas import tpu as pltpu
from jax.experimental.pallas import tpu_sc as plsc

_NUM_CORES = 2
_NUM_SUBCORES = 16
_NUM_WORKERS = _NUM_CORES * _NUM_SUBCORES
_ROWS_PER_CHUNK = 4


def kernel(token_ids, weights):
    batch, seq = token_ids.shape
    num_idx = batch * seq
    num_rows, dim = weights.shape

    idx = token_ids.reshape(num_idx)
    half = lax.shift_right_logical(idx, 1)
    wpair = weights.reshape(num_rows // 2, 2 * dim)

    rows_per_w = batch // _NUM_WORKERS
    n_chunks = rows_per_w // _ROWS_PER_CHUNK
    chunk_idx = _ROWS_PER_CHUNK * seq

    mesh = plsc.VectorSubcoreMesh(core_axis_name="c", subcore_axis_name="s")

    @functools.partial(
        pl.kernel,
        mesh=mesh,
        out_type=jax.ShapeDtypeStruct((batch, seq, 2 * dim), weights.dtype),
        scratch_types=[
            pltpu.VMEM((128,), jnp.int32),
            pltpu.VMEM((chunk_idx - 128,), jnp.int32),
            pltpu.VMEM((chunk_idx, 2 * dim), jnp.float32),
            pltpu.SemaphoreType.DMA,
        ],
    )
    def gather_kernel(table_hbm, ih_hbm, out_hbm, iva, ivb, rows_v, sem):
        wid = lax.axis_index("s") * _NUM_CORES + lax.axis_index("c")
        row_base = wid * rows_per_w

        @pl.loop(0, n_chunks)
        def _(c):
            row0 = row_base + c * _ROWS_PER_CHUNK
            off = row0 * seq
            pltpu.sync_copy(ih_hbm.at[pl.ds(off, 128)], iva)
            pltpu.sync_copy(ih_hbm.at[pl.ds(off + 128, chunk_idx - 128)], ivb)
            ga = pltpu.async_copy(
                table_hbm.at[iva], rows_v.at[pl.ds(0, 128)], sem
            )
            gb = pltpu.async_copy(
                table_hbm.at[ivb], rows_v.at[pl.ds(128, chunk_idx - 128)], sem
            )
            ga.wait()
            gb.wait()
            for r in range(_ROWS_PER_CHUNK):
                pltpu.sync_copy(
                    rows_v.at[pl.ds(r * seq, seq)], out_hbm.at[row0 + r]
                )

    pairs = gather_kernel(wpair, half)
    odd = lax.bitwise_and(token_ids, 1)[..., None] == 1
    return jnp.where(odd, pairs[..., dim:], pairs[..., :dim])


# speedup vs baseline: 1.5299x; 1.1464x over previous
import functools

import jax
import jax.numpy as jnp
from jax import lax
from jax.experimental import pallas as pl
from jax.experimental.pallas import tpu as pltpu
from jax.experimental.pallas import tpu_sc as plsc

_NUM_CORES = 2
_NUM_SUBCORES = 16
_NUM_WORKERS = _NUM_CORES * _NUM_SUBCORES
_RPC = 4
_SLOTS = 4


def kernel(token_ids, weights):
    batch, seq = token_ids.shape
    num_idx = batch * seq
    num_rows, dim = weights.shape

    idx = token_ids.reshape(num_idx)
    half = lax.shift_right_logical(idx, 1)
    wpair = weights.reshape(num_rows // 2, 2 * dim)

    rows_per_w = batch // _NUM_WORKERS
    n_chunks = rows_per_w // _RPC
    chunk_idx = _RPC * seq
    na = 128
    nb = chunk_idx - na

    mesh = plsc.VectorSubcoreMesh(core_axis_name="c", subcore_axis_name="s")

    scratch = (
        [pltpu.VMEM((chunk_idx, 2 * dim), jnp.float32) for _ in range(_SLOTS)]
        + [pltpu.VMEM((na,), jnp.int32) for _ in range(_SLOTS)]
        + [pltpu.VMEM((nb,), jnp.int32) for _ in range(_SLOTS)]
        + [pltpu.SemaphoreType.DMA for _ in range(3 * _SLOTS)]
    )

    @functools.partial(
        pl.kernel,
        mesh=mesh,
        out_type=jax.ShapeDtypeStruct((batch, seq, 2 * dim), weights.dtype),
        scratch_types=scratch,
    )
    def gather_kernel(table_hbm, ih_hbm, out_hbm, *scr):
        rvs = scr[0:4]
        iva = scr[4:8]
        ivb = scr[8:12]
        isem = scr[12:16]
        gsem = scr[16:20]
        ssem = scr[20:24]

        wid = lax.axis_index("s") * _NUM_CORES + lax.axis_index("c")
        row_base = wid * rows_per_w

        def idx_copies(t, s):
            off = (row_base + t * _RPC) * seq
            return (
                pltpu.make_async_copy(ih_hbm.at[pl.ds(off, na)], iva[s], isem[s]),
                pltpu.make_async_copy(
                    ih_hbm.at[pl.ds(off + na, nb)], ivb[s], isem[s]
                ),
            )

        def gather_copies(t, s):
            return (
                pltpu.make_async_copy(
                    table_hbm.at[iva[s]], rvs[s].at[pl.ds(0, na)], gsem[s]
                ),
                pltpu.make_async_copy(
                    table_hbm.at[ivb[s]], rvs[s].at[pl.ds(na, nb)], gsem[s]
                ),
            )

        def store_copies(t, s):
            row0 = row_base + t * _RPC
            return tuple(
                pltpu.make_async_copy(
                    rvs[s].at[pl.ds(r * seq, seq)], out_hbm.at[row0 + r], ssem[s]
                )
                for r in range(_RPC)
            )

        def start(cs):
            for c in cs:
                c.start()

        def wait(cs):
            for c in cs:
                c.wait()

        start(idx_copies(0, 0))
        for t in range(_SLOTS):
            s = t
            wait(idx_copies(t, s))
            start(gather_copies(t, s))
            start(idx_copies(t + 1, (t + 1) % _SLOTS))
            if t >= 1:
                wait(gather_copies(t - 1, s - 1))
                start(store_copies(t - 1, s - 1))

        @pl.loop(1, n_chunks // _SLOTS - 1)
        def _(k):
            t0 = k * _SLOTS
            for j in range(_SLOTS):
                t = t0 + j
                s = j
                pj = (j - 1) % _SLOTS
                wait(idx_copies(t, s))
                wait(store_copies(t - _SLOTS, s))
                start(gather_copies(t, s))
                start(idx_copies(t + 1, (j + 1) % _SLOTS))
                wait(gather_copies(t - 1, pj))
                start(store_copies(t - 1, pj))

        for j in range(_SLOTS):
            t = n_chunks - _SLOTS + j
            s = j
            pj = (j - 1) % _SLOTS
            wait(idx_copies(t, s))
            wait(store_copies(t - _SLOTS, s))
            start(gather_copies(t, s))
            if t + 1 < n_chunks:
                start(idx_copies(t + 1, (j + 1) % _SLOTS))
            wait(gather_copies(t - 1, pj))
            start(store_copies(t - 1, pj))

        wait(gather_copies(n_chunks - 1, _SLOTS - 1))
        start(store_copies(n_chunks - 1, _SLOTS - 1))
        for j in range(_SLOTS):
            wait(store_copies(n_chunks - _SLOTS + j, j))

    pairs = gather_kernel(wpair, half)
    odd = lax.bitwise_and(token_ids, 1)[..., None] == 1
    return jnp.where(odd, pairs[..., dim:], pairs[..., :dim])
